# baseline (speedup 1.0000x reference)
import functools

import jax
import jax.numpy as jnp
from jax import lax
from jax.experimental import pallas as pl
from jax.experimental.pallas import tpu as pltpu
from jax.experimental.pallas import tpu_sc as plsc

N, D, E = 10000, 128, 320000
NC, NS = 2, 16
NW = NC * NS
B = 80
CPT = 125
EP = NW * B * CPT
NBUF = 8
NP = 10016
RPS = NP // NS
DEGW = 16

_mesh = plsc.VectorSubcoreMesh(
    core_axis_name="c", subcore_axis_name="s", num_cores=NC, num_subcores=NS)
_sc_params = pltpu.CompilerParams(use_tc_tiling_on_sc=False)


@functools.partial(
    pl.kernel,
    out_type=jax.ShapeDtypeStruct((NC, NP, DEGW), jnp.float32),
    mesh=_mesh,
    scratch_types=[
        pltpu.VMEM((B, DEGW), jnp.float32),
        pltpu.VMEM((CPT, B), jnp.int32),
        pltpu.VMEM_SHARED((NP, DEGW), jnp.float32),
        pltpu.SemaphoreType.DMA,
    ],
    compiler_params=_sc_params,
)
def _deg_sc(dst_hbm, ones_hbm, zeros_hbm, out_hbm, ones_v, dst_v, acc, sem):
    c = lax.axis_index("c")
    s = lax.axis_index("s")
    wid = c * NS + s
    pltpu.sync_copy(zeros_hbm.at[pl.ds(s * RPS, RPS)], acc.at[pl.ds(s * RPS, RPS)])
    pltpu.sync_copy(ones_hbm, ones_v)
    pltpu.sync_copy(dst_hbm.at[pl.ds(wid * CPT, CPT)], dst_v)
    plsc.subcore_barrier()

    LAG = 8

    def body(j, carry):
        @pl.when(j < CPT)
        def _():
            pltpu.async_copy(ones_v, acc.at[dst_v.at[j]], sem, add=True)

        @pl.when(j >= LAG)
        def _():
            pltpu.make_async_copy(ones_v, acc.at[dst_v.at[j - LAG]], sem).wait()

        return carry

    lax.fori_loop(0, CPT + LAG, body, 0)
    plsc.subcore_barrier()
    pltpu.sync_copy(acc.at[pl.ds(s * RPS, RPS)],
                    out_hbm.at[c, pl.ds(s * RPS, RPS)])


@functools.partial(
    pl.kernel,
    out_type=jax.ShapeDtypeStruct((NC, NP, D), jnp.bfloat16),
    mesh=_mesh,
    scratch_types=(
        [pltpu.VMEM((CPT, B), jnp.int32),
         pltpu.VMEM((CPT, B), jnp.int32)]
        + [pltpu.VMEM((B, D), jnp.bfloat16) for _ in range(NBUF)]
        + [pltpu.VMEM_SHARED((NP, D), jnp.bfloat16)]
        + [pltpu.SemaphoreType.DMA for _ in range(2 * NBUF)]
    ),
    compiler_params=_sc_params,
)
def _agg_sc(g_hbm, src_hbm, dst_hbm, zeros_hbm, out_hbm,
            src_v, dst_v, *rest):
    bufs = rest[:NBUF]
    acc = rest[NBUF]
    gsem = rest[NBUF + 1:NBUF + 1 + NBUF]
    ssem = rest[NBUF + 1 + NBUF:]
    c = lax.axis_index("c")
    s = lax.axis_index("s")
    wid = c * NS + s
    d0 = pltpu.async_copy(
        zeros_hbm.at[pl.ds(s * RPS, RPS)], acc.at[pl.ds(s * RPS, RPS)], gsem[0])
    d1 = pltpu.async_copy(src_hbm.at[pl.ds(wid * CPT, CPT)], src_v, gsem[1])
    d2 = pltpu.async_copy(dst_hbm.at[pl.ds(wid * CPT, CPT)], dst_v, ssem[0])
    d0.wait()
    d1.wait()
    d2.wait()
    plsc.subcore_barrier()

    def issue_gather(j, b):
        pltpu.async_copy(g_hbm.at[src_v.at[j]], bufs[b], gsem[b])

    def wait_gather(j, b):
        pltpu.make_async_copy(g_hbm.at[src_v.at[j]], bufs[b], gsem[b]).wait()

    def issue_scatter(j, b):
        pltpu.async_copy(bufs[b], acc.at[dst_v.at[j]], ssem[b], add=True)

    def wait_scatter(j, b):
        pltpu.make_async_copy(bufs[b], acc.at[dst_v.at[j]], ssem[b]).wait()

    A = NBUF - 1
    for j in range(A):
        issue_gather(j, j)

    def body(o, carry):
        for b in range(NBUF):
            j = o * NBUF + b

            @pl.when(j < CPT)
            def _():
                wait_gather(j, b)
                issue_scatter(j, b)

            @pl.when(jnp.logical_and(j >= 1, j <= CPT))
            def _():
                wait_scatter(j - 1, (b + A) % NBUF)

            @pl.when(j + A < CPT)
            def _():
                issue_gather(j + A, (b + A) % NBUF)
        return carry

    lax.fori_loop(0, (CPT + NBUF) // NBUF, body, 0)

    plsc.subcore_barrier()
    pltpu.sync_copy(acc.at[pl.ds(s * RPS, RPS)],
                    out_hbm.at[c, pl.ds(s * RPS, RPS)])


NB = 25
RB = N // NB

_blk = pl.BlockSpec((RB, D), lambda i: (i, 0))
_blks = pl.BlockSpec((NC, RB, D), lambda i: (0, i, 0))
_blkdeg0 = pl.BlockSpec((1, RB, DEGW), lambda i: (0, i, 0))
_blkdeg1 = pl.BlockSpec((1, RB, DEGW), lambda i: (1, i, 0))
_blkw = pl.BlockSpec((D, D), lambda i: (0, 0))
_blkb = pl.BlockSpec((1, D), lambda i: (0, 0))
_tc_params = pltpu.CompilerParams(dimension_semantics=("parallel",))


def _dinv_of(dp0_ref, dp1_ref):
    return lax.rsqrt(1.0 + dp0_ref[0, :, 0:1] + dp1_ref[0, :, 0:1])


def _tc_first_body(dp0_ref, dp1_ref, x_ref, w_ref, o_ref, ob_ref):
    dinv = _dinv_of(dp0_ref, dp1_ref)
    h = jnp.dot(x_ref[...], w_ref[...], preferred_element_type=jnp.float32)
    g = h * dinv
    o_ref[...] = g
    ob_ref[...] = g.astype(jnp.bfloat16)


def _tc_mid_body(dp0_ref, dp1_ref, g_ref, s_ref, w_ref, b_ref, o_ref, ob_ref):
    dinv = _dinv_of(dp0_ref, dp1_ref)
    sagg = s_ref[0].astype(jnp.float32) + s_ref[1].astype(jnp.float32)
    z = dinv * (g_ref[...] + sagg) + b_ref[...]
    z = jnp.maximum(z, 0.0)
    h = jnp.dot(z, w_ref[...], preferred_element_type=jnp.float32)
    g = h * dinv
    o_ref[...] = g
    ob_ref[...] = g.astype(jnp.bfloat16)


def _tc_last_body(dp0_ref, dp1_ref, g_ref, s_ref, w_ref, b_ref, bh_ref, o_ref):
    dinv = _dinv_of(dp0_ref, dp1_ref)
    sagg = s_ref[0].astype(jnp.float32) + s_ref[1].astype(jnp.float32)
    z = dinv * (g_ref[...] + sagg) + b_ref[...]
    h = jnp.dot(z, w_ref[...], preferred_element_type=jnp.float32)
    o_ref[...] = h + bh_ref[...]


_out_f32 = jax.ShapeDtypeStruct((N, D), jnp.float32)
_out_bf16 = jax.ShapeDtypeStruct((N, D), jnp.bfloat16)

_tc_first = pl.pallas_call(
    _tc_first_body,
    grid=(NB,),
    in_specs=[_blkdeg0, _blkdeg1, _blk, _blkw],
    out_specs=[_blk, _blk],
    out_shape=[_out_f32, _out_bf16],
    compiler_params=_tc_params,
)

_tc_mid = pl.pallas_call(
    _tc_mid_body,
    grid=(NB,),
    in_specs=[_blkdeg0, _blkdeg1, _blk, _blks, _blkw, _blkb],
    out_specs=[_blk, _blk],
    out_shape=[_out_f32, _out_bf16],
    compiler_params=_tc_params,
)

_tc_last = pl.pallas_call(
    _tc_last_body,
    grid=(NB,),
    in_specs=[_blkdeg0, _blkdeg1, _blk, _blks, _blkw, _blkb, _blkb],
    out_specs=_blk,
    out_shape=_out_f32,
    compiler_params=_tc_params,
)


def kernel(x, edge_index, W1, b1, W2, b2, W3, b3, Wh, bh):
    src2 = edge_index[0].reshape(EP // B, B)
    dst2 = edge_index[1].reshape(EP // B, B)
    zeros_d = jnp.zeros((NP, D), jnp.bfloat16)
    zeros_w = jnp.zeros((NP, DEGW), jnp.float32)
    ones_w = jnp.ones((B, DEGW), jnp.float32)
    b1r = b1.reshape(1, D)
    b2r = b2.reshape(1, D)
    b3r = b3.reshape(1, D)
    bhr = bh.reshape(1, D)

    degp = _deg_sc(dst2, ones_w, zeros_w)

    g1, g1b = _tc_first(degp, degp, x, W1)
    s1 = _agg_sc(g1b, src2, dst2, zeros_d)
    g2, g2b = _tc_mid(degp, degp, g1, s1, W2, b1r)
    s2 = _agg_sc(g2b, src2, dst2, zeros_d)
    g3, g3b = _tc_mid(degp, degp, g2, s2, W3, b2r)
    s3 = _agg_sc(g3b, src2, dst2, zeros_d)
    out = _tc_last(degp, degp, g3, s3, Wh, b3r, bhr)
    return out


# speedup vs baseline: 1.1178x; 1.1178x over previous
import functools

import jax
import jax.numpy as jnp
from jax import lax
from jax.experimental import pallas as pl
from jax.experimental.pallas import tpu as pltpu
from jax.experimental.pallas import tpu_sc as plsc

N, D, E = 10000, 128, 320000
NC, NS = 2, 16
NW = NC * NS
B = 80
CPT = 125
EP = NW * B * CPT
NBUF = 8
NP = 10016
RPS = NP // NS
DEGW = 16

_mesh = plsc.VectorSubcoreMesh(
    core_axis_name="c", subcore_axis_name="s", num_cores=NC, num_subcores=NS)
_sc_params = pltpu.CompilerParams(use_tc_tiling_on_sc=False)


@functools.partial(
    pl.kernel,
    out_type=jax.ShapeDtypeStruct((NC, NP, DEGW), jnp.float32),
    mesh=_mesh,
    scratch_types=[
        pltpu.VMEM((B, DEGW), jnp.float32),
        pltpu.VMEM((CPT, B), jnp.int32),
        pltpu.VMEM_SHARED((NP, DEGW), jnp.float32),
        pltpu.SemaphoreType.DMA,
    ],
    compiler_params=_sc_params,
)
def _deg_sc(dst_hbm, ones_hbm, zeros_hbm, out_hbm, ones_v, dst_v, acc, sem):
    c = lax.axis_index("c")
    s = lax.axis_index("s")
    wid = c * NS + s
    pltpu.sync_copy(zeros_hbm.at[pl.ds(s * RPS, RPS)], acc.at[pl.ds(s * RPS, RPS)])
    pltpu.sync_copy(ones_hbm, ones_v)
    pltpu.sync_copy(dst_hbm.at[pl.ds(wid * CPT, CPT)], dst_v)
    plsc.subcore_barrier()

    LAG = 8

    def body(j, carry):
        @pl.when(j < CPT)
        def _():
            pltpu.async_copy(ones_v, acc.at[dst_v.at[j]], sem, add=True)

        @pl.when(j >= LAG)
        def _():
            pltpu.make_async_copy(ones_v, acc.at[dst_v.at[j - LAG]], sem).wait()

        return carry

    lax.fori_loop(0, CPT + LAG, body, 0)
    plsc.subcore_barrier()
    pltpu.sync_copy(acc.at[pl.ds(s * RPS, RPS)],
                    out_hbm.at[c, pl.ds(s * RPS, RPS)])


@functools.partial(
    pl.kernel,
    out_type=jax.ShapeDtypeStruct((NC, NP, D), jnp.bfloat16),
    mesh=_mesh,
    scratch_types=(
        [pltpu.VMEM((CPT, B), jnp.int32),
         pltpu.VMEM((CPT, B), jnp.int32)]
        + [pltpu.VMEM((B, D), jnp.bfloat16) for _ in range(NBUF)]
        + [pltpu.VMEM_SHARED((NP, D), jnp.bfloat16)]
        + [pltpu.SemaphoreType.DMA for _ in range(2 * NBUF)]
    ),
    compiler_params=_sc_params,
)
def _agg_sc(g_hbm, src_hbm, dst_hbm, zeros_hbm, out_hbm,
            src_v, dst_v, *rest):
    bufs = rest[:NBUF]
    acc = rest[NBUF]
    gsem = rest[NBUF + 1:NBUF + 1 + NBUF]
    ssem = rest[NBUF + 1 + NBUF:]
    c = lax.axis_index("c")
    s = lax.axis_index("s")
    wid = c * NS + s
    d0 = pltpu.async_copy(
        zeros_hbm.at[pl.ds(s * RPS, RPS)], acc.at[pl.ds(s * RPS, RPS)], gsem[0])
    d1 = pltpu.async_copy(src_hbm.at[pl.ds(wid * CPT, CPT)], src_v, gsem[1])
    d2 = pltpu.async_copy(dst_hbm.at[pl.ds(wid * CPT, CPT)], dst_v, ssem[0])
    d0.wait()
    d1.wait()
    d2.wait()
    plsc.subcore_barrier()

    def issue_gather(j, b):
        pltpu.async_copy(g_hbm.at[src_v.at[j]], bufs[b], gsem[b])

    def wait_gather(j, b):
        pltpu.make_async_copy(g_hbm.at[src_v.at[j]], bufs[b], gsem[b]).wait()

    def issue_scatter(j, b):
        pltpu.async_copy(bufs[b], acc.at[dst_v.at[j]], ssem[b], add=True)

    def wait_scatter(j, b):
        pltpu.make_async_copy(bufs[b], acc.at[dst_v.at[j]], ssem[b]).wait()

    A = NBUF - 1
    for j in range(A):
        issue_gather(j, j)

    def body(o, carry):
        for b in range(NBUF):
            j = o * NBUF + b

            @pl.when(j < CPT)
            def _():
                wait_gather(j, b)
                issue_scatter(j, b)

            @pl.when(jnp.logical_and(j >= 1, j <= CPT))
            def _():
                wait_scatter(j - 1, (b + A) % NBUF)

            @pl.when(j + A < CPT)
            def _():
                issue_gather(j + A, (b + A) % NBUF)
        return carry

    lax.fori_loop(0, (CPT + NBUF) // NBUF, body, 0)

    plsc.subcore_barrier()
    pltpu.sync_copy(acc.at[pl.ds(s * RPS, RPS)],
                    out_hbm.at[c, pl.ds(s * RPS, RPS)])


NB = 5
RB = N // NB

_blk = pl.BlockSpec((RB, D), lambda i: (i, 0))
_blks = pl.BlockSpec((NC, RB, D), lambda i: (0, i, 0))
_blkdeg0 = pl.BlockSpec((1, RB, DEGW), lambda i: (0, i, 0))
_blkdeg1 = pl.BlockSpec((1, RB, DEGW), lambda i: (1, i, 0))
_blkw = pl.BlockSpec((D, D), lambda i: (0, 0))
_blkb = pl.BlockSpec((1, D), lambda i: (0, 0))
_tc_params = pltpu.CompilerParams(dimension_semantics=("parallel",))


def _dinv_of(dp0_ref, dp1_ref):
    return lax.rsqrt(1.0 + dp0_ref[0, :, 0:1] + dp1_ref[0, :, 0:1])


def _tc_first_body(dp0_ref, dp1_ref, x_ref, w_ref, o_ref, ob_ref):
    dinv = _dinv_of(dp0_ref, dp1_ref)
    h = jnp.dot(x_ref[...], w_ref[...], preferred_element_type=jnp.float32)
    g = h * dinv
    o_ref[...] = g
    ob_ref[...] = g.astype(jnp.bfloat16)


def _tc_mid_body(dp0_ref, dp1_ref, g_ref, s_ref, w_ref, b_ref, o_ref, ob_ref):
    dinv = _dinv_of(dp0_ref, dp1_ref)
    sagg = s_ref[0].astype(jnp.float32) + s_ref[1].astype(jnp.float32)
    z = dinv * (g_ref[...] + sagg) + b_ref[...]
    z = jnp.maximum(z, 0.0)
    h = jnp.dot(z, w_ref[...], preferred_element_type=jnp.float32)
    g = h * dinv
    o_ref[...] = g
    ob_ref[...] = g.astype(jnp.bfloat16)


def _tc_last_body(dp0_ref, dp1_ref, g_ref, s_ref, w_ref, b_ref, bh_ref, o_ref):
    dinv = _dinv_of(dp0_ref, dp1_ref)
    sagg = s_ref[0].astype(jnp.float32) + s_ref[1].astype(jnp.float32)
    z = dinv * (g_ref[...] + sagg) + b_ref[...]
    h = jnp.dot(z, w_ref[...], preferred_element_type=jnp.float32)
    o_ref[...] = h + bh_ref[...]


_out_f32 = jax.ShapeDtypeStruct((N, D), jnp.float32)
_out_bf16 = jax.ShapeDtypeStruct((N, D), jnp.bfloat16)

_tc_first = pl.pallas_call(
    _tc_first_body,
    grid=(NB,),
    in_specs=[_blkdeg0, _blkdeg1, _blk, _blkw],
    out_specs=[_blk, _blk],
    out_shape=[_out_f32, _out_bf16],
    compiler_params=_tc_params,
)

_tc_mid = pl.pallas_call(
    _tc_mid_body,
    grid=(NB,),
    in_specs=[_blkdeg0, _blkdeg1, _blk, _blks, _blkw, _blkb],
    out_specs=[_blk, _blk],
    out_shape=[_out_f32, _out_bf16],
    compiler_params=_tc_params,
)

_tc_last = pl.pallas_call(
    _tc_last_body,
    grid=(NB,),
    in_specs=[_blkdeg0, _blkdeg1, _blk, _blks, _blkw, _blkb, _blkb],
    out_specs=_blk,
    out_shape=_out_f32,
    compiler_params=_tc_params,
)


def kernel(x, edge_index, W1, b1, W2, b2, W3, b3, Wh, bh):
    src2 = edge_index[0].reshape(EP // B, B)
    dst2 = edge_index[1].reshape(EP // B, B)
    zeros_d = jnp.zeros((NP, D), jnp.bfloat16)
    zeros_w = jnp.zeros((NP, DEGW), jnp.float32)
    ones_w = jnp.ones((B, DEGW), jnp.float32)
    b1r = b1.reshape(1, D)
    b2r = b2.reshape(1, D)
    b3r = b3.reshape(1, D)
    bhr = bh.reshape(1, D)

    degp = _deg_sc(dst2, ones_w, zeros_w)

    g1, g1b = _tc_first(degp, degp, x, W1)
    s1 = _agg_sc(g1b, src2, dst2, zeros_d)
    g2, g2b = _tc_mid(degp, degp, g1, s1, W2, b1r)
    s2 = _agg_sc(g2b, src2, dst2, zeros_d)
    g3, g3b = _tc_mid(degp, degp, g2, s2, W3, b2r)
    s3 = _agg_sc(g3b, src2, dst2, zeros_d)
    out = _tc_last(degp, degp, g3, s3, Wh, b3r, bhr)
    return out
